# baseline (speedup 1.0000x reference)
import jax
import jax.numpy as jnp
from jax.experimental import pallas as pl
from jax.experimental.pallas import tpu as pltpu

EMB = 32
N = 4096
BLK = 512
NBLK = N // BLK


def _mmT(x, w):
    return jax.lax.dot_general(x, w, (((1,), (1,)), ((), ())),
                               preferred_element_type=jnp.float32)


def _dot(a, b):
    return jnp.dot(a, b, preferred_element_type=jnp.float32)


def _body(graph_ref, xv_ref, w1t_ref, w2_ref, w3_ref, w4t_ref, w5a_ref,
          w5b_ref, w6_ref, w7_ref, out_ref, gb_ref, ebf_a, ebf_b,
          c_ref, acc_ref, r6w_ref):
    p = pl.program_id(0)
    k = pl.program_id(1)
    row = pl.ds(k * BLK, BLK)

    @pl.when((p == 0) & (k == 0))
    def _zero():
        ebf_a[...] = jnp.zeros((N, EMB), jnp.bfloat16)
        acc_ref[...] = jnp.zeros((N, EMB), jnp.float32)

    @pl.when(p == 0)
    def _init():
        r = jnp.zeros((BLK, 1), jnp.float32)
        for j in range(NBLK):
            gj = graph_ref[:, j * BLK:(j + 1) * BLK]
            gb_ref[row, j * BLK:(j + 1) * BLK] = gj.astype(jnp.bfloat16)
            r = r + jnp.sum(gj, axis=1, keepdims=True)
        a = xv_ref[row, :] * w1t_ref[...]
        ut = _mmT(w4t_ref[...], w3_ref[...])
        cb = a + r * ut
        c_ref[row, :] = cb
        e1 = jnp.maximum(cb, 0.0).astype(jnp.bfloat16)
        ebf_a[row, :] = e1
        for j in range(NBLK):
            @pl.when(j <= k)
            def _aj(j=j):
                part = _dot(gb_ref[row, j * BLK:(j + 1) * BLK],
                            ebf_a[j * BLK:(j + 1) * BLK, :])
                acc_ref[row, :] += part
        for i in range(NBLK):
            @pl.when(i < k)
            def _bi(i=i):
                gslice = gb_ref[i * BLK:(i + 1) * BLK, pl.ds(k * BLK, BLK)]
                acc_ref[i * BLK:(i + 1) * BLK, :] += _dot(gslice, e1)

    @pl.when(p == 1)
    def _fin1():
        v2 = _mmT(acc_ref[row, :], w2_ref[...])
        ebf_b[row, :] = jnp.maximum(c_ref[row, :] + v2, 0.0).astype(jnp.bfloat16)

    def _step(src, dst):
        gb = gb_ref[row, :]
        ns = _dot(gb, src[...])
        v2 = _mmT(ns, w2_ref[...])
        e = jnp.maximum(c_ref[row, :] + v2, 0.0)
        dst[row, :] = e.astype(jnp.bfloat16)

    pl.when(p == 2)(lambda: _step(ebf_b, ebf_a))
    pl.when(p == 3)(lambda: _step(ebf_a, ebf_b))

    @pl.when((p == 4) & (k == 0))
    def _glob():
        es = jnp.sum(ebf_b[...].astype(jnp.float32), axis=0, keepdims=True)
        r6 = jnp.maximum(_mmT(es, w6_ref[...]), 0.0)
        r6w_ref[...] = r6 * w5a_ref[...]

    @pl.when(p == 4)
    def _out():
        r7 = jnp.maximum(_mmT(ebf_b[row, :], w7_ref[...]), 0.0)
        out_ref[...] = jnp.sum(r7 * w5b_ref[...] + r6w_ref[...],
                               axis=1, keepdims=True)


def kernel(graph, Xv, W1, W2, W3, W4, W5, W6, W7):
    g2 = graph.reshape(N, N)
    xv2 = Xv.reshape(N, 1)
    w1t = W1.reshape(1, EMB)
    w4t = W4.reshape(1, EMB)
    w5a = W5[:, :EMB]
    w5b = W5[:, EMB:]

    full = lambda shape: pl.BlockSpec(shape, lambda p, i: (0, 0))
    out = pl.pallas_call(
        _body,
        grid=(5, NBLK),
        in_specs=[
            pl.BlockSpec((BLK, N), lambda p, i: (jnp.where(p == 0, i, NBLK - 1), 0)),
            full((N, 1)),
            full((1, EMB)),
            full((EMB, EMB)),
            full((EMB, EMB)),
            full((1, EMB)),
            full((1, EMB)),
            full((1, EMB)),
            full((EMB, EMB)),
            full((EMB, EMB)),
        ],
        out_specs=pl.BlockSpec((BLK, 1), lambda p, i: (jnp.where(p == 4, i, 0), 0)),
        out_shape=jax.ShapeDtypeStruct((N, 1), jnp.float32),
        scratch_shapes=[
            pltpu.VMEM((N, N), jnp.bfloat16),
            pltpu.VMEM((N, EMB), jnp.bfloat16),
            pltpu.VMEM((N, EMB), jnp.bfloat16),
            pltpu.VMEM((N, EMB), jnp.float32),
            pltpu.VMEM((N, EMB), jnp.float32),
            pltpu.VMEM((1, EMB), jnp.float32),
        ],
        compiler_params=pltpu.CompilerParams(
            dimension_semantics=("arbitrary", "arbitrary")),
    )(g2, xv2, w1t, W2, W3, w4t, w5a, w5b, W6, W7)
    return out.reshape(1, N)
